# baseline (speedup 1.0000x reference)
import functools

import jax
import jax.numpy as jnp
from jax import lax
from jax.experimental import pallas as pl
from jax.experimental.pallas import tpu as pltpu
from jax.experimental.pallas import tpu_sc as plsc

EPS = 1e-5


def _sc_gather(table, idx_flat):
    num_rows = idx_flat.shape[0]
    d = table.shape[1]
    info = plsc.get_sparse_core_info()
    nw = info.num_cores * info.num_subcores
    rows_per_w = num_rows // nw
    chunk = 1280
    while rows_per_w % chunk:
        chunk //= 2
    n_chunks = rows_per_w // chunk

    mesh = plsc.VectorSubcoreMesh(core_axis_name="c", subcore_axis_name="s")

    @functools.partial(
        pl.kernel,
        mesh=mesh,
        compiler_params=pltpu.CompilerParams(use_tc_tiling_on_sc=False),
        out_type=jax.ShapeDtypeStruct((num_rows, d), jnp.float32),
        scratch_types=[
            pltpu.VMEM((chunk,), jnp.int32),
            pltpu.VMEM((chunk, d), jnp.float32),
            pltpu.SemaphoreType.DMA,
        ],
    )
    def gather_kernel(table_hbm, idx_hbm, out_hbm, idx_v, rows_v, sem):
        wid = lax.axis_index("s") * info.num_cores + lax.axis_index("c")
        base = wid * rows_per_w

        def body(ci, carry):
            start = pl.multiple_of(base + ci * chunk, 8)
            pltpu.sync_copy(idx_hbm.at[pl.ds(start, chunk)], idx_v)
            pltpu.async_copy(table_hbm.at[idx_v], rows_v, sem).wait()
            pltpu.sync_copy(rows_v, out_hbm.at[pl.ds(start, chunk)])
            return carry

        lax.fori_loop(0, n_chunks, body, 0)

    return gather_kernel(table, idx_flat)


def _tc_body(emb_ref, pos_ref, gamma_ref, beta_ref, wp2_ref, out_ref):
    rows2 = emb_ref.shape[0]
    half = rows2 // pos_ref.shape[0]
    d = gamma_ref.shape[-1]
    for j in range(pos_ref.shape[0]):
        sl = slice(j * half, (j + 1) * half)
        e3 = (emb_ref[sl, :] + pos_ref[j]).reshape(half, 2, d)
        mu = jnp.mean(e3, axis=2, keepdims=True)
        var = jnp.mean((e3 - mu) ** 2, axis=2, keepdims=True)
        hn3 = (e3 - mu) * lax.rsqrt(var + EPS)
        hn3 = hn3 * gamma_ref[...] + beta_ref[...]
        hn = hn3.reshape(half, 2 * d)
        out_ref[sl, :] = jnp.dot(hn, wp2_ref[...], preferred_element_type=jnp.float32)


def kernel(x, table, pos_table, gamma, beta, Wp):
    b, s = x.shape
    d = table.shape[1]
    m = Wp.shape[1]
    num_rows = b * s

    idx_sm = jnp.swapaxes(x, 0, 1).reshape(num_rows)
    emb = _sc_gather(table, idx_sm)
    emb2 = emb.reshape(num_rows // 2, 2 * d)

    blk = 8192
    s_per_blk = blk // b

    pos2 = jnp.concatenate([pos_table, pos_table], axis=1)
    wp2 = jnp.zeros((2 * d, 2 * m), Wp.dtype)
    wp2 = wp2.at[:d, :m].set(Wp).at[d:, m:].set(Wp)

    out = pl.pallas_call(
        _tc_body,
        grid=(num_rows // blk,),
        compiler_params=pltpu.CompilerParams(vmem_limit_bytes=62914560),
        in_specs=[
            pl.BlockSpec((blk // 2, 2 * d), lambda i: (i, 0)),
            pl.BlockSpec((s_per_blk, 1, 2 * d), lambda i: (i, 0, 0)),
            pl.BlockSpec((1, 1, d), lambda i: (0, 0, 0)),
            pl.BlockSpec((1, 1, d), lambda i: (0, 0, 0)),
            pl.BlockSpec((2 * d, 2 * m), lambda i: (0, 0)),
        ],
        out_specs=pl.BlockSpec((blk // 2, 2 * m), lambda i: (i, 0)),
        out_shape=jax.ShapeDtypeStruct((num_rows // 2, 2 * m), jnp.float32),
    )(emb2, pos2.reshape(-1, 1, 2 * d), gamma.reshape(1, 1, d),
      beta.reshape(1, 1, d), wp2)

    return jnp.swapaxes(out.reshape(s, b, m), 0, 1)


# speedup vs baseline: 1.5674x; 1.5674x over previous
import functools

import jax
import jax.numpy as jnp
from jax import lax
from jax.experimental import pallas as pl
from jax.experimental.pallas import tpu as pltpu
from jax.experimental.pallas import tpu_sc as plsc

EPS = 1e-5


def _sc_gather(table, idx_flat):
    num_rows = idx_flat.shape[0]
    d = table.shape[1]
    info = plsc.get_sparse_core_info()
    nw = info.num_cores * info.num_subcores
    rows_per_w = num_rows // nw
    chunk = 1280
    while rows_per_w % chunk:
        chunk //= 2
    n_chunks = rows_per_w // chunk

    mesh = plsc.VectorSubcoreMesh(core_axis_name="c", subcore_axis_name="s")

    @functools.partial(
        pl.kernel,
        mesh=mesh,
        compiler_params=pltpu.CompilerParams(use_tc_tiling_on_sc=False),
        out_type=jax.ShapeDtypeStruct((num_rows, d), jnp.float32),
        scratch_types=[
            pltpu.VMEM((chunk,), jnp.int32),
            pltpu.VMEM((chunk, d), jnp.float32),
            pltpu.SemaphoreType.DMA,
        ],
    )
    def gather_kernel(table_hbm, idx_hbm, out_hbm, idx_v, rows_v, sem):
        wid = lax.axis_index("s") * info.num_cores + lax.axis_index("c")
        base = wid * rows_per_w

        def body(ci, carry):
            start = pl.multiple_of(base + ci * chunk, 8)
            pltpu.sync_copy(idx_hbm.at[pl.ds(start, chunk)], idx_v)
            pltpu.async_copy(table_hbm.at[idx_v], rows_v, sem).wait()
            pltpu.sync_copy(rows_v, out_hbm.at[pl.ds(start, chunk)])
            return carry

        lax.fori_loop(0, n_chunks, body, 0)

    return gather_kernel(table, idx_flat)


def _tc_body(emb_ref, pos_ref, gamma_ref, beta_ref, wp_ref, out_ref):
    half = emb_ref.shape[0] // pos_ref.shape[0]
    for j in range(pos_ref.shape[0]):
        sl = pl.ds(j * half, half)
        h = emb_ref[sl, :] + pos_ref[j]
        mu = jnp.mean(h, axis=1, keepdims=True)
        var = jnp.mean((h - mu) ** 2, axis=1, keepdims=True)
        hn = (h - mu) * lax.rsqrt(var + EPS)
        hn = hn * gamma_ref[...] + beta_ref[...]
        out_ref[sl, :] = jnp.dot(hn, wp_ref[...], preferred_element_type=jnp.float32)


def kernel(x, table, pos_table, gamma, beta, Wp):
    b, s = x.shape
    d = table.shape[1]
    m = Wp.shape[1]
    num_rows = b * s

    idx_sm = jnp.swapaxes(x, 0, 1).reshape(num_rows)
    emb = _sc_gather(table, idx_sm)

    blk = 8192
    s_per_blk = blk // b

    out = pl.pallas_call(
        _tc_body,
        grid=(num_rows // blk,),
        in_specs=[
            pl.BlockSpec((blk, d), lambda i: (i, 0)),
            pl.BlockSpec((s_per_blk, 1, d), lambda i: (i, 0, 0)),
            pl.BlockSpec((1, d), lambda i: (0, 0)),
            pl.BlockSpec((1, d), lambda i: (0, 0)),
            pl.BlockSpec((d, m), lambda i: (0, 0)),
        ],
        out_specs=pl.BlockSpec((blk, m), lambda i: (i, 0)),
        out_shape=jax.ShapeDtypeStruct((num_rows, m), jnp.float32),
    )(emb, pos_table.reshape(-1, 1, d), gamma.reshape(1, d), beta.reshape(1, d), Wp)

    return jnp.swapaxes(out.reshape(s, b, m), 0, 1)


# speedup vs baseline: 1.5695x; 1.0013x over previous
import functools

import jax
import jax.numpy as jnp
from jax import lax
from jax.experimental import pallas as pl
from jax.experimental.pallas import tpu as pltpu
from jax.experimental.pallas import tpu_sc as plsc

EPS = 1e-5


def _sc_gather(table, idx_flat):
    num_rows = idx_flat.shape[0]
    d = table.shape[1]
    info = plsc.get_sparse_core_info()
    nw = info.num_cores * info.num_subcores
    rows_per_w = num_rows // nw
    chunk = 1280
    while rows_per_w % chunk:
        chunk //= 2
    n_chunks = rows_per_w // chunk

    mesh = plsc.VectorSubcoreMesh(core_axis_name="c", subcore_axis_name="s")

    @functools.partial(
        pl.kernel,
        mesh=mesh,
        compiler_params=pltpu.CompilerParams(use_tc_tiling_on_sc=False),
        out_type=jax.ShapeDtypeStruct((num_rows, d), jnp.float32),
        scratch_types=[
            pltpu.VMEM((rows_per_w,), jnp.int32),
            pltpu.VMEM((chunk, d), jnp.float32),
            pltpu.SemaphoreType.DMA,
        ],
    )
    def gather_kernel(table_hbm, idx_hbm, out_hbm, idx_v, rows_v, sem):
        wid = lax.axis_index("s") * info.num_cores + lax.axis_index("c")
        base = wid * rows_per_w
        pltpu.sync_copy(idx_hbm.at[pl.ds(pl.multiple_of(base, 8), rows_per_w)], idx_v)

        def body(ci, carry):
            off = ci * chunk
            start = pl.multiple_of(base + off, 8)
            pltpu.async_copy(
                table_hbm.at[idx_v.at[pl.ds(off, chunk)]], rows_v, sem
            ).wait()
            pltpu.sync_copy(rows_v, out_hbm.at[pl.ds(start, chunk)])
            return carry

        lax.fori_loop(0, n_chunks, body, 0)

    return gather_kernel(table, idx_flat)


def _tc_body(emb_ref, pos_ref, gamma_ref, beta_ref, wp_ref, out_ref):
    half = emb_ref.shape[0] // pos_ref.shape[0]
    for j in range(pos_ref.shape[0]):
        sl = pl.ds(j * half, half)
        h = emb_ref[sl, :] + pos_ref[j]
        mu = jnp.mean(h, axis=1, keepdims=True)
        var = jnp.mean((h - mu) ** 2, axis=1, keepdims=True)
        hn = (h - mu) * lax.rsqrt(var + EPS)
        hn = hn * gamma_ref[...] + beta_ref[...]
        out_ref[sl, :] = jnp.dot(hn, wp_ref[...], preferred_element_type=jnp.float32)


def kernel(x, table, pos_table, gamma, beta, Wp):
    b, s = x.shape
    d = table.shape[1]
    m = Wp.shape[1]
    num_rows = b * s

    idx_sm = jnp.swapaxes(x, 0, 1).reshape(num_rows)
    emb = _sc_gather(table, idx_sm)

    blk = 8192
    s_per_blk = blk // b

    out = pl.pallas_call(
        _tc_body,
        grid=(num_rows // blk,),
        in_specs=[
            pl.BlockSpec((blk, d), lambda i: (i, 0)),
            pl.BlockSpec((s_per_blk, 1, d), lambda i: (i, 0, 0)),
            pl.BlockSpec((1, d), lambda i: (0, 0)),
            pl.BlockSpec((1, d), lambda i: (0, 0)),
            pl.BlockSpec((d, m), lambda i: (0, 0)),
        ],
        out_specs=pl.BlockSpec((blk, m), lambda i: (i, 0)),
        out_shape=jax.ShapeDtypeStruct((num_rows, m), jnp.float32),
    )(emb, pos_table.reshape(-1, 1, d), gamma.reshape(1, d), beta.reshape(1, d), Wp)

    return jnp.swapaxes(out.reshape(s, b, m), 0, 1)
